# Initial kernel scaffold; baseline (speedup 1.0000x reference)
import jax
import jax.numpy as jnp
from jax.experimental import pallas as pl


def kernel(z, a, mu, tau):
    raise NotImplementedError("write your pallas kernel here")


# speedup vs baseline: 1.6438x; 1.6438x over previous
import functools

import jax
import jax.numpy as jnp
from jax import lax
from jax.experimental import pallas as pl
from jax.experimental.pallas import tpu as pltpu
from jax.experimental.pallas import tpu_sc as plsc

FEAT = 128
BATCH = 16384
MOM = 0.3

_info = plsc.get_sparse_core_info()
_NC, _NS = _info.num_cores, _info.num_subcores
_NW = _NC * _NS
_B_PER_W = BATCH // _NW
_CHUNK = 128
_NCHUNK = _B_PER_W // _CHUNK


def _sc_gather_body(idx_hbm, mu_hbm, tau_hbm, mu_out, tau_out,
                    idx_v, mu_v, tau_v, sem_mu, sem_tau):
    wid = lax.axis_index("s") * _NC + lax.axis_index("c")
    base = wid * _B_PER_W
    for c in range(_NCHUNK):
        off = base + c * _CHUNK
        pltpu.sync_copy(idx_hbm.at[pl.ds(off, _CHUNK)], idx_v)
        cp_mu = pltpu.async_copy(mu_hbm.at[idx_v], mu_v, sem_mu)
        cp_tau = pltpu.async_copy(tau_hbm.at[idx_v], tau_v, sem_tau)
        cp_mu.wait()
        cp_tau.wait()
        pltpu.sync_copy(mu_v, mu_out.at[pl.ds(off, _CHUNK)])
        pltpu.sync_copy(tau_v, tau_out.at[pl.ds(off, _CHUNK)])


_sc_gather = functools.partial(
    pl.kernel,
    mesh=plsc.VectorSubcoreMesh(core_axis_name="c", subcore_axis_name="s"),
    out_type=[
        jax.ShapeDtypeStruct((BATCH, FEAT), jnp.float32),
        jax.ShapeDtypeStruct((BATCH, FEAT), jnp.float32),
    ],
    scratch_types=[
        pltpu.VMEM((_CHUNK,), jnp.int32),
        pltpu.VMEM((_CHUNK, FEAT), jnp.float32),
        pltpu.VMEM((_CHUNK, FEAT), jnp.float32),
        pltpu.SemaphoreType.DMA,
        pltpu.SemaphoreType.DMA,
    ],
)(_sc_gather_body)


def _tc_norm_body(z_ref, mu_ref, tau_ref, o_ref):
    z = z_ref[...]
    sigma = jnp.log1p(jnp.exp(tau_ref[...]))
    z_hat = (z - mu_ref[...]) / sigma
    o_ref[...] = (1.0 - MOM) * z_hat + MOM * z


def _tc_norm(z, mu_a, tau_a):
    blk = 2048
    grid = (BATCH // blk,)
    spec = pl.BlockSpec((blk, FEAT), lambda i: (i, 0))
    return pl.pallas_call(
        _tc_norm_body,
        grid=grid,
        in_specs=[spec, spec, spec],
        out_specs=spec,
        out_shape=jax.ShapeDtypeStruct((BATCH, FEAT), jnp.float32),
    )(z, mu_a, tau_a)


def kernel(z, a, mu, tau):
    a32 = a.astype(jnp.int32)
    mu_a, tau_a = _sc_gather(a32, mu, tau)
    return _tc_norm(z, mu_a, tau_a)


# speedup vs baseline: 1.6903x; 1.0283x over previous
import functools

import jax
import jax.numpy as jnp
from jax import lax
from jax.experimental import pallas as pl
from jax.experimental.pallas import tpu as pltpu
from jax.experimental.pallas import tpu_sc as plsc

FEAT = 128
BATCH = 16384
MOM = 0.3

_info = plsc.get_sparse_core_info()
_NC, _NS = _info.num_cores, _info.num_subcores
_NW = _NC * _NS
_B_PER_W = BATCH // _NW
_CHUNK = 128
_NCHUNK = _B_PER_W // _CHUNK


def _sc_gather_body(idx_hbm, mu_hbm, tau_hbm, mu_out, tau_out,
                    idx0, idx1, idx2, idx3,
                    mu_a, tau_a, mu_b, tau_b,
                    sem_ga, sem_gb, sem_sa, sem_sb):
    wid = lax.axis_index("s") * _NC + lax.axis_index("c")
    base = wid * _B_PER_W
    idx_bufs = (idx0, idx1, idx2, idx3)
    for c in range(_NCHUNK):
        pltpu.sync_copy(idx_hbm.at[pl.ds(base + c * _CHUNK, _CHUNK)],
                        idx_bufs[c])

    def gather(c, mu_v, tau_v, sem):
        cp0 = pltpu.async_copy(mu_hbm.at[idx_bufs[c]], mu_v, sem)
        cp1 = pltpu.async_copy(tau_hbm.at[idx_bufs[c]], tau_v, sem)
        return cp0, cp1

    def scatter(c, mu_v, tau_v, sem):
        off = base + c * _CHUNK
        cp0 = pltpu.async_copy(mu_v, mu_out.at[pl.ds(off, _CHUNK)], sem)
        cp1 = pltpu.async_copy(tau_v, tau_out.at[pl.ds(off, _CHUNK)], sem)
        return cp0, cp1

    g0 = gather(0, mu_a, tau_a, sem_ga)
    g1 = gather(1, mu_b, tau_b, sem_gb)
    g0[0].wait(); g0[1].wait()
    s0 = scatter(0, mu_a, tau_a, sem_sa)
    g1[0].wait(); g1[1].wait()
    s1 = scatter(1, mu_b, tau_b, sem_sb)
    s0[0].wait(); s0[1].wait()
    g2 = gather(2, mu_a, tau_a, sem_ga)
    s1[0].wait(); s1[1].wait()
    g3 = gather(3, mu_b, tau_b, sem_gb)
    g2[0].wait(); g2[1].wait()
    s2 = scatter(2, mu_a, tau_a, sem_sa)
    g3[0].wait(); g3[1].wait()
    s3 = scatter(3, mu_b, tau_b, sem_sb)
    s2[0].wait(); s2[1].wait()
    s3[0].wait(); s3[1].wait()


_sc_gather = functools.partial(
    pl.kernel,
    mesh=plsc.VectorSubcoreMesh(core_axis_name="c", subcore_axis_name="s"),
    out_type=[
        jax.ShapeDtypeStruct((BATCH, FEAT), jnp.float32),
        jax.ShapeDtypeStruct((BATCH, FEAT), jnp.float32),
    ],
    scratch_types=[
        pltpu.VMEM((_CHUNK,), jnp.int32),
        pltpu.VMEM((_CHUNK,), jnp.int32),
        pltpu.VMEM((_CHUNK,), jnp.int32),
        pltpu.VMEM((_CHUNK,), jnp.int32),
        pltpu.VMEM((_CHUNK, FEAT), jnp.float32),
        pltpu.VMEM((_CHUNK, FEAT), jnp.float32),
        pltpu.VMEM((_CHUNK, FEAT), jnp.float32),
        pltpu.VMEM((_CHUNK, FEAT), jnp.float32),
        pltpu.SemaphoreType.DMA,
        pltpu.SemaphoreType.DMA,
        pltpu.SemaphoreType.DMA,
        pltpu.SemaphoreType.DMA,
    ],
)(_sc_gather_body)


def _tc_norm_body(z_ref, mu_ref, tau_ref, o_ref):
    z = z_ref[...]
    sigma = jnp.log1p(jnp.exp(tau_ref[...]))
    z_hat = (z - mu_ref[...]) / sigma
    o_ref[...] = (1.0 - MOM) * z_hat + MOM * z


def _tc_norm(z, mu_a, tau_a):
    blk = 2048
    grid = (BATCH // blk,)
    spec = pl.BlockSpec((blk, FEAT), lambda i: (i, 0))
    return pl.pallas_call(
        _tc_norm_body,
        grid=grid,
        in_specs=[spec, spec, spec],
        out_specs=spec,
        out_shape=jax.ShapeDtypeStruct((BATCH, FEAT), jnp.float32),
    )(z, mu_a, tau_a)


def kernel(z, a, mu, tau):
    a32 = a.astype(jnp.int32)
    mu_a, tau_a = _sc_gather(a32, mu, tau)
    return _tc_norm(z, mu_a, tau_a)
